# baseline (speedup 1.0000x reference)
import functools

import jax
import jax.numpy as jnp
import numpy as np
from jax import lax
from jax.experimental import pallas as pl
from jax.experimental.pallas import tpu as pltpu
from jax.experimental.pallas import tpu_sc as plsc

N = 10000
E = 160000
F = 10
H = 8
D = 64

NC = 2
NS = 16
NW = NC * NS
L = 16

CHUNK = 128
NCH = 40
E_PAD = NW * NCH * CHUNK
NROW = E_PAD // CHUNK
R = 10112
RPT = R // NS
WACC = 96
WOUT = 128
SRCW = 24
ERW = 16

_hp = np.zeros((6, 16), np.int32)
_fp = np.zeros((6, 16), np.int32)
for _j in range(6):
    for _l in range(16):
        _col = 16 * _j + _l
        if _col < 8:
            _hp[_j, _l] = _col
            _fp[_j, _l] = 15
        else:
            _k = _col - 8
            if _k < 80:
                _hp[_j, _l] = _k // 10
                _fp[_j, _l] = _k % 10
            else:
                _hp[_j, _l] = 0
                _fp[_j, _l] = 10
_PATS = np.concatenate([_hp, _fp], axis=0)


def _xlane_gather(x, idx):
    dn = lax.GatherDimensionNumbers(
        offset_dims=(), collapsed_slice_dims=(0,), start_index_map=(0,))
    return lax.gather(x, idx[:, None], dn, (1,),
                      mode=lax.GatherScatterMode.PROMISE_IN_BOUNDS)


def _tables_body(f1_ref, f2_ref, asrc_ref, adst_ref, ei_ref,
                 st_ref, er_ref, si_ref, di_ref):
    f1 = f1_ref[...]
    f2 = f2_ref[...]
    el = jnp.dot(f1, asrc_ref[...], preferred_element_type=jnp.float32)
    er = jnp.dot(f2, adst_ref[...], preferred_element_type=jnp.float32)
    z8 = jnp.zeros((N, 8), jnp.float32)
    z5 = jnp.zeros((N, 5), jnp.float32)
    o1 = jnp.ones((N, 1), jnp.float32)
    st_ref[pl.ds(0, N), :] = jnp.concatenate([el, f1, z5, o1], axis=1)
    st_ref[pl.ds(N, R - N), :] = jnp.zeros((R - N, SRCW), jnp.float32)
    er_ref[pl.ds(0, N), :] = jnp.concatenate([er, z8], axis=1)
    er_ref[pl.ds(N, R - N), :] = jnp.zeros((R - N, ERW), jnp.float32)
    sfull = jnp.concatenate(
        [ei_ref[0, :], jnp.zeros((E_PAD - E,), jnp.int32)])
    dfull = jnp.concatenate(
        [ei_ref[1, :], jnp.full((E_PAD - E,), R - 1, jnp.int32)])
    si_ref[...] = sfull.reshape(NROW, CHUNK)
    di_ref[...] = dfull.reshape(NROW, CHUNK)


def _build_tables(feat1, feat2, a_src, a_dst, ei32):
    return pl.pallas_call(
        _tables_body,
        out_shape=[
            jax.ShapeDtypeStruct((R, SRCW), jnp.float32),
            jax.ShapeDtypeStruct((R, ERW), jnp.float32),
            jax.ShapeDtypeStruct((NROW, CHUNK), jnp.int32),
            jax.ShapeDtypeStruct((NROW, CHUNK), jnp.int32),
        ],
    )(feat1, feat2, a_src, a_dst, ei32)


def _sc_body(st_hbm, er_hbm, sidx_hbm, didx_hbm, pat_hbm, z_hbm, out_hbm,
             accum, sidx, didx, srows, erows, orows, pats,
             sem_s0, sem_s1, sem_e0, sem_e1, sem_o0, sem_o1):
    c = lax.axis_index("c")
    s = lax.axis_index("s")
    wid = c * NS + s
    row0 = s * RPT
    sem_s = (sem_s0, sem_s1)
    sem_e = (sem_e0, sem_e1)
    sem_o = (sem_o0, sem_o1)

    pltpu.sync_copy(z_hbm.at[pl.ds(row0, RPT)], accum.at[pl.ds(row0, RPT)])
    pltpu.sync_copy(sidx_hbm.at[pl.ds(wid * NCH, NCH)], sidx)
    pltpu.sync_copy(didx_hbm.at[pl.ds(wid * NCH, NCH)], didx)
    pltpu.sync_copy(pat_hbm, pats)
    pv0 = tuple(pats[j, :] for j in range(12))
    plsc.subcore_barrier()

    def start_gather(ch, b):
        pltpu.async_copy(st_hbm.at[sidx.at[ch]], srows.at[b], sem_s[b])
        pltpu.async_copy(er_hbm.at[didx.at[ch]], erows.at[b], sem_e[b])

    def wait_gather(ch, b):
        pltpu.make_async_copy(st_hbm.at[sidx.at[ch]], srows.at[b],
                              sem_s[b]).wait()
        pltpu.make_async_copy(er_hbm.at[didx.at[ch]], erows.at[b],
                              sem_e[b]).wait()

    def start_scatter(ch, b):
        pltpu.async_copy(orows.at[b], accum.at[didx.at[ch]], sem_o[b],
                         add=True)

    def wait_scatter(ch, b):
        pltpu.make_async_copy(orows.at[b], accum.at[didx.at[ch]],
                              sem_o[b]).wait()

    def compute_chunk(b, pv):
        @plsc.parallel_loop(0, CHUNK, step=1, unroll=8, carry=pv)
        def body(e, pvc):
            vel = srows[b, e, pl.ds(0, 16)]
            vft = srows[b, e, pl.ds(8, 16)]
            ver = erows[b, e, pl.ds(0, 16)]
            x = vel + ver
            x = jnp.maximum(x, 0.2 * x)
            w = jnp.exp(x)
            for j in range(6):
                wv = _xlane_gather(w, pvc[j])
                fv = _xlane_gather(vft, pvc[6 + j])
                orows[b, e, pl.ds(16 * j, 16)] = wv * fv
            return pvc
        return body

    start_gather(0, 0)

    def pair_body(i, pv):
        for b in (0, 1):
            ch = 2 * i + b

            wait_gather(ch, b)

            @pl.when(ch + 1 < NCH)
            def _():
                start_gather(ch + 1, 1 - b)

            @pl.when(ch >= 2)
            def _():
                wait_scatter(ch, b)

            pv = compute_chunk(b, pv)
            start_scatter(ch, b)
        return pv

    lax.fori_loop(0, NCH // 2, pair_body, pv0)
    wait_scatter(NCH - 2, 0)
    wait_scatter(NCH - 1, 1)
    plsc.subcore_barrier()
    pltpu.sync_copy(accum.at[pl.ds(row0, RPT)],
                    out_hbm.at[c, pl.ds(row0, RPT), pl.ds(0, WACC)])


_sc_edge = functools.partial(
    pl.kernel,
    out_type=jax.ShapeDtypeStruct((NC, R, WOUT), jnp.float32),
    mesh=plsc.VectorSubcoreMesh(core_axis_name="c", subcore_axis_name="s"),
    compiler_params=pltpu.CompilerParams(use_tc_tiling_on_sc=False),
    scratch_types=[
        pltpu.VMEM_SHARED((R, WACC), jnp.float32),
        pltpu.VMEM((NCH, CHUNK), jnp.int32),
        pltpu.VMEM((NCH, CHUNK), jnp.int32),
        pltpu.VMEM((2, CHUNK, SRCW), jnp.float32),
        pltpu.VMEM((2, CHUNK, ERW), jnp.float32),
        pltpu.VMEM((2, CHUNK, WACC), jnp.float32),
        pltpu.VMEM((12, 16), jnp.int32),
        pltpu.SemaphoreType.DMA,
        pltpu.SemaphoreType.DMA,
        pltpu.SemaphoreType.DMA,
        pltpu.SemaphoreType.DMA,
        pltpu.SemaphoreType.DMA,
        pltpu.SemaphoreType.DMA,
    ],
)(_sc_body)


BLK = 2048


def _finish_body(p_ref, w3t_ref, b3_ref, o_ref):
    combt = jnp.transpose(p_ref[0] + p_ref[1])
    den = combt[0:8, :]
    recip = 1.0 / jnp.where(den > 0.0, den, 1.0)
    for h in range(H):
        ut = combt[8 + F * h:8 + F * (h + 1), :]
        gt = ut * recip[h:h + 1, :]
        o_ref[h] = (jnp.dot(w3t_ref[h], gt,
                            preferred_element_type=jnp.float32)
                    + b3_ref[h][:, None])


def _finish(partials, w3t, bias3):
    return pl.pallas_call(
        _finish_body,
        grid=(pl.cdiv(N, BLK),),
        in_specs=[
            pl.BlockSpec((NC, BLK, WOUT), lambda i: (0, i, 0)),
            pl.BlockSpec((H, D, F), lambda i: (0, 0, 0)),
            pl.BlockSpec((H, D), lambda i: (0, 0)),
        ],
        out_specs=pl.BlockSpec((H, D, BLK), lambda i: (0, 0, i)),
        out_shape=jax.ShapeDtypeStruct((H, D, N), jnp.float32),
    )(partials, w3t, bias3)


def kernel(feat, edge_index, feat1, feat2, W_src, W_dst, attn_l, attn_r, bias):
    del feat
    ei32 = edge_index.astype(jnp.int32)

    W3s = W_src.reshape(F, H, D)
    a_src = jnp.einsum("fhd,hd->fh", W3s, attn_l)
    a_dst = jnp.einsum("fhd,hd->fh", W_dst.reshape(F, H, D), attn_r)
    w3t = W3s.transpose(1, 2, 0)
    bias3 = bias.reshape(H, D)

    src_table, er_table, sidx, didx = _build_tables(
        feat1, feat2, a_src, a_dst, ei32)

    pats = jnp.asarray(_PATS)
    zeros = jnp.zeros((R, WACC), jnp.float32)
    partials = _sc_edge(src_table, er_table, sidx, didx, pats, zeros)

    out_t = _finish(partials, w3t, bias3)
    return jnp.transpose(out_t, (2, 0, 1))


# speedup vs baseline: 1.0029x; 1.0029x over previous
import functools

import jax
import jax.numpy as jnp
import numpy as np
from jax import lax
from jax.experimental import pallas as pl
from jax.experimental.pallas import tpu as pltpu
from jax.experimental.pallas import tpu_sc as plsc

N = 10000
E = 160000
F = 10
H = 8
D = 64

NC = 2
NS = 16
NW = NC * NS
L = 16

CHUNK = 128
NCH = 40
E_PAD = NW * NCH * CHUNK
NROW = E_PAD // CHUNK
R = 10112
RPT = R // NS
WACC = 96
WOUT = 128
SRCW = 24
ERW = 16

_hp = np.zeros((6, 16), np.int32)
_fp = np.zeros((6, 16), np.int32)
for _j in range(6):
    for _l in range(16):
        _col = 16 * _j + _l
        if _col < 8:
            _hp[_j, _l] = _col
            _fp[_j, _l] = 15
        else:
            _k = _col - 8
            if _k < 80:
                _hp[_j, _l] = _k // 10
                _fp[_j, _l] = _k % 10
            else:
                _hp[_j, _l] = 0
                _fp[_j, _l] = 10
_PATS = np.concatenate([_hp, _fp], axis=0)


def _xlane_gather(x, idx):
    dn = lax.GatherDimensionNumbers(
        offset_dims=(), collapsed_slice_dims=(0,), start_index_map=(0,))
    return lax.gather(x, idx[:, None], dn, (1,),
                      mode=lax.GatherScatterMode.PROMISE_IN_BOUNDS)


def _tables_body(f1_ref, f2_ref, asrc_ref, adst_ref, ei_ref,
                 st_ref, er_ref, si_ref, di_ref):
    f1 = f1_ref[...]
    f2 = f2_ref[...]
    el = jnp.dot(f1, asrc_ref[...], preferred_element_type=jnp.float32)
    er = jnp.dot(f2, adst_ref[...], preferred_element_type=jnp.float32)
    z8 = jnp.zeros((N, 8), jnp.float32)
    z5 = jnp.zeros((N, 5), jnp.float32)
    o1 = jnp.ones((N, 1), jnp.float32)
    st_ref[pl.ds(0, N), :] = jnp.concatenate([el, f1, z5, o1], axis=1)
    st_ref[pl.ds(N, R - N), :] = jnp.zeros((R - N, SRCW), jnp.float32)
    er_ref[pl.ds(0, N), :] = jnp.concatenate([er, z8], axis=1)
    er_ref[pl.ds(N, R - N), :] = jnp.zeros((R - N, ERW), jnp.float32)
    sfull = jnp.concatenate(
        [ei_ref[0, :], jnp.zeros((E_PAD - E,), jnp.int32)])
    dfull = jnp.concatenate(
        [ei_ref[1, :], jnp.full((E_PAD - E,), R - 1, jnp.int32)])
    si_ref[...] = sfull.reshape(NROW, CHUNK)
    di_ref[...] = dfull.reshape(NROW, CHUNK)


def _build_tables(feat1, feat2, a_src, a_dst, ei32):
    return pl.pallas_call(
        _tables_body,
        out_shape=[
            jax.ShapeDtypeStruct((R, SRCW), jnp.float32),
            jax.ShapeDtypeStruct((R, ERW), jnp.float32),
            jax.ShapeDtypeStruct((NROW, CHUNK), jnp.int32),
            jax.ShapeDtypeStruct((NROW, CHUNK), jnp.int32),
        ],
    )(feat1, feat2, a_src, a_dst, ei32)


def _sc_body(st_hbm, er_hbm, sidx_hbm, didx_hbm, pat_hbm, z_hbm, out_hbm,
             accum, sidx, didx, srows, erows, orows, pats,
             sem_s0, sem_s1, sem_e0, sem_e1, sem_o0, sem_o1):
    c = lax.axis_index("c")
    s = lax.axis_index("s")
    wid = c * NS + s
    row0 = s * RPT
    sem_s = (sem_s0, sem_s1)
    sem_e = (sem_e0, sem_e1)
    sem_o = (sem_o0, sem_o1)

    pltpu.sync_copy(z_hbm.at[pl.ds(row0, RPT)], accum.at[pl.ds(row0, RPT)])
    pltpu.sync_copy(sidx_hbm.at[pl.ds(wid * NCH, NCH)], sidx)
    pltpu.sync_copy(didx_hbm.at[pl.ds(wid * NCH, NCH)], didx)
    pltpu.sync_copy(pat_hbm, pats)
    pv0 = tuple(pats[j, :] for j in range(12))
    plsc.subcore_barrier()

    HALF = CHUNK // 2

    def start_gather(ch, b):
        for hf in (0, 1):
            pltpu.async_copy(
                st_hbm.at[sidx.at[ch].at[pl.ds(hf * HALF, HALF)]],
                srows.at[b].at[pl.ds(hf * HALF, HALF)], sem_s[b])
            pltpu.async_copy(
                er_hbm.at[didx.at[ch].at[pl.ds(hf * HALF, HALF)]],
                erows.at[b].at[pl.ds(hf * HALF, HALF)], sem_e[b])

    def wait_gather(ch, b):
        for hf in (0, 1):
            pltpu.make_async_copy(
                st_hbm.at[sidx.at[ch].at[pl.ds(hf * HALF, HALF)]],
                srows.at[b].at[pl.ds(hf * HALF, HALF)], sem_s[b]).wait()
            pltpu.make_async_copy(
                er_hbm.at[didx.at[ch].at[pl.ds(hf * HALF, HALF)]],
                erows.at[b].at[pl.ds(hf * HALF, HALF)], sem_e[b]).wait()

    def start_scatter(ch, b):
        pltpu.async_copy(orows.at[b], accum.at[didx.at[ch]], sem_o[b],
                         add=True)

    def wait_scatter(ch, b):
        pltpu.make_async_copy(orows.at[b], accum.at[didx.at[ch]],
                              sem_o[b]).wait()

    def compute_chunk(b, pv):
        @plsc.parallel_loop(0, CHUNK, step=1, unroll=4, carry=pv)
        def body(e, pvc):
            vel = srows[b, e, pl.ds(0, 16)]
            vft = srows[b, e, pl.ds(8, 16)]
            ver = erows[b, e, pl.ds(0, 16)]
            x = vel + ver
            x = jnp.maximum(x, 0.2 * x)
            w = jnp.exp(x)
            for j in range(6):
                wv = _xlane_gather(w, pvc[j])
                fv = _xlane_gather(vft, pvc[6 + j])
                orows[b, e, pl.ds(16 * j, 16)] = wv * fv
            return pvc
        return body

    start_gather(0, 0)

    def pair_body(i, pv):
        for b in (0, 1):
            ch = 2 * i + b

            wait_gather(ch, b)

            @pl.when(ch + 1 < NCH)
            def _():
                start_gather(ch + 1, 1 - b)

            @pl.when(ch >= 2)
            def _():
                wait_scatter(ch, b)

            pv = compute_chunk(b, pv)
            start_scatter(ch, b)
        return pv

    lax.fori_loop(0, NCH // 2, pair_body, pv0)
    wait_scatter(NCH - 2, 0)
    wait_scatter(NCH - 1, 1)
    plsc.subcore_barrier()
    pltpu.sync_copy(accum.at[pl.ds(row0, RPT)],
                    out_hbm.at[c, pl.ds(row0, RPT), pl.ds(0, WACC)])


_sc_edge = functools.partial(
    pl.kernel,
    out_type=jax.ShapeDtypeStruct((NC, R, WOUT), jnp.float32),
    mesh=plsc.VectorSubcoreMesh(core_axis_name="c", subcore_axis_name="s"),
    compiler_params=pltpu.CompilerParams(use_tc_tiling_on_sc=False),
    scratch_types=[
        pltpu.VMEM_SHARED((R, WACC), jnp.float32),
        pltpu.VMEM((NCH, CHUNK), jnp.int32),
        pltpu.VMEM((NCH, CHUNK), jnp.int32),
        pltpu.VMEM((2, CHUNK, SRCW), jnp.float32),
        pltpu.VMEM((2, CHUNK, ERW), jnp.float32),
        pltpu.VMEM((2, CHUNK, WACC), jnp.float32),
        pltpu.VMEM((12, 16), jnp.int32),
        pltpu.SemaphoreType.DMA,
        pltpu.SemaphoreType.DMA,
        pltpu.SemaphoreType.DMA,
        pltpu.SemaphoreType.DMA,
        pltpu.SemaphoreType.DMA,
        pltpu.SemaphoreType.DMA,
    ],
)(_sc_body)


BLK = 2048


def _finish_body(p_ref, w3t_ref, b3_ref, o_ref):
    combt = jnp.transpose(p_ref[0] + p_ref[1])
    den = combt[0:8, :]
    recip = 1.0 / jnp.where(den > 0.0, den, 1.0)
    for h in range(H):
        ut = combt[8 + F * h:8 + F * (h + 1), :]
        gt = ut * recip[h:h + 1, :]
        o_ref[h] = (jnp.dot(w3t_ref[h], gt,
                            preferred_element_type=jnp.float32)
                    + b3_ref[h][:, None])


def _finish(partials, w3t, bias3):
    return pl.pallas_call(
        _finish_body,
        grid=(pl.cdiv(N, BLK),),
        in_specs=[
            pl.BlockSpec((NC, BLK, WOUT), lambda i: (0, i, 0)),
            pl.BlockSpec((H, D, F), lambda i: (0, 0, 0)),
            pl.BlockSpec((H, D), lambda i: (0, 0)),
        ],
        out_specs=pl.BlockSpec((H, D, BLK), lambda i: (0, 0, i)),
        out_shape=jax.ShapeDtypeStruct((H, D, N), jnp.float32),
    )(partials, w3t, bias3)


def kernel(feat, edge_index, feat1, feat2, W_src, W_dst, attn_l, attn_r, bias):
    del feat
    ei32 = edge_index.astype(jnp.int32)

    W3s = W_src.reshape(F, H, D)
    a_src = jnp.einsum("fhd,hd->fh", W3s, attn_l)
    a_dst = jnp.einsum("fhd,hd->fh", W_dst.reshape(F, H, D), attn_r)
    w3t = W3s.transpose(1, 2, 0)
    bias3 = bias.reshape(H, D)

    src_table, er_table, sidx, didx = _build_tables(
        feat1, feat2, a_src, a_dst, ei32)

    pats = jnp.asarray(_PATS)
    zeros = jnp.zeros((R, WACC), jnp.float32)
    partials = _sc_edge(src_table, er_table, sidx, didx, pats, zeros)

    out_t = _finish(partials, w3t, bias3)
    return jnp.transpose(out_t, (2, 0, 1))
